# Initial kernel scaffold; baseline (speedup 1.0000x reference)
import jax
import jax.numpy as jnp
from jax.experimental import pallas as pl


def kernel(pairs, gmf_user, gmf_item, mlp_user, mlp_item, W1, b1, W2, b2, W3, b3, W4, b4, Wh, bh):
    raise NotImplementedError("write your pallas kernel here")


# speedup vs baseline: 2.8833x; 2.8833x over previous
import functools

import jax
import jax.numpy as jnp
from jax import lax
from jax.experimental import pallas as pl
from jax.experimental.pallas import tpu as pltpu
from jax.experimental.pallas import tpu_sc as plsc

B = 16384
DIM = 128
NC = 2
NS = 16
NW = NC * NS
BPW = B // NW
CHUNK = 128
NCHUNK = BPW // CHUNK


def _sc_gather_body(users_hbm, items_hbm, gu_t, gi_t, mu_t, mi_t,
                    gu_o, gi_o, mu_o, mi_o,
                    idxu, idxi, bu, bi, bmu, bmi, sem):
    wid = lax.axis_index("s") * NC + lax.axis_index("c")
    base = wid * BPW
    for c in range(NCHUNK):
        off = base + c * CHUNK
        pltpu.sync_copy(users_hbm.at[pl.ds(off, CHUNK)], idxu)
        pltpu.sync_copy(items_hbm.at[pl.ds(off, CHUNK)], idxi)
        d0 = pltpu.async_copy(gu_t.at[idxu], bu, sem)
        d1 = pltpu.async_copy(gi_t.at[idxi], bi, sem)
        d2 = pltpu.async_copy(mu_t.at[idxu], bmu, sem)
        d3 = pltpu.async_copy(mi_t.at[idxi], bmi, sem)
        d0.wait(); d1.wait(); d2.wait(); d3.wait()
        pltpu.sync_copy(bu, gu_o.at[pl.ds(off, CHUNK)])
        pltpu.sync_copy(bi, gi_o.at[pl.ds(off, CHUNK)])
        pltpu.sync_copy(bmu, mu_o.at[pl.ds(off, CHUNK)])
        pltpu.sync_copy(bmi, mi_o.at[pl.ds(off, CHUNK)])


def _sc_gather(users, items, gmf_user, gmf_item, mlp_user, mlp_item):
    mesh = plsc.VectorSubcoreMesh(
        core_axis_name="c", subcore_axis_name="s",
        num_cores=NC, num_subcores=NS)
    row = jax.ShapeDtypeStruct((B, DIM), jnp.float32)
    fn = pl.kernel(
        _sc_gather_body,
        out_type=(row, row, row, row),
        mesh=mesh,
        scratch_types=[
            pltpu.VMEM((CHUNK,), jnp.int32),
            pltpu.VMEM((CHUNK,), jnp.int32),
            pltpu.VMEM((CHUNK, DIM), jnp.float32),
            pltpu.VMEM((CHUNK, DIM), jnp.float32),
            pltpu.VMEM((CHUNK, DIM), jnp.float32),
            pltpu.VMEM((CHUNK, DIM), jnp.float32),
            pltpu.SemaphoreType.DMA,
        ],
    )
    return fn(users, items, gmf_user, gmf_item, mlp_user, mlp_item)


BT = 2048


def _tc_mlp_body(gu, gi, mu, mi, w1a, w1b, w2, w3, w4, wha, whb,
                 b1, b2, b3, b4, bh, out_ref):
    f32 = jnp.float32
    h = jnp.dot(mu[...], w1a[...], preferred_element_type=f32)
    h += jnp.dot(mi[...], w1b[...], preferred_element_type=f32)
    h = jnp.maximum(h + b1[...], 0.0)
    h = jnp.maximum(jnp.dot(h, w2[...], preferred_element_type=f32) + b2[...], 0.0)
    h = jnp.maximum(jnp.dot(h, w3[...], preferred_element_type=f32) + b3[...], 0.0)
    y2 = jnp.maximum(jnp.dot(h, w4[...], preferred_element_type=f32) + b4[...], 0.0)
    s1 = jnp.sum(gu[...] * gi[...] * wha[...], axis=1, keepdims=True)
    s2 = jnp.dot(y2, whb[...], preferred_element_type=f32)
    out_ref[...] = jax.nn.sigmoid(s1 + s2 + bh[...])


def _tc_mlp(gu, gi, mu, mi, w1a, w1b, w2, w3, w4, wha, whb, b1, b2, b3, b4, bh):
    grid = (B // BT,)
    row_spec = pl.BlockSpec((BT, DIM), lambda i: (i, 0))

    def _full(a):
        return pl.BlockSpec(a.shape, lambda i: tuple(0 for _ in a.shape))

    small = [w1a, w1b, w2, w3, w4, wha, whb, b1, b2, b3, b4, bh]
    return pl.pallas_call(
        _tc_mlp_body,
        grid=grid,
        in_specs=[row_spec, row_spec, row_spec, row_spec] + [_full(a) for a in small],
        out_specs=pl.BlockSpec((BT, 1), lambda i: (i, 0)),
        out_shape=jax.ShapeDtypeStruct((B, 1), jnp.float32),
        compiler_params=pltpu.CompilerParams(
            dimension_semantics=("arbitrary",)),
    )(gu, gi, mu, mi, *small)


def kernel(pairs, gmf_user, gmf_item, mlp_user, mlp_item,
           W1, b1, W2, b2, W3, b3, W4, b4, Wh, bh):
    users = pairs[:, 0].astype(jnp.int32)
    items = pairs[:, 1].astype(jnp.int32)
    gu, gi, mu, mi = _sc_gather(users, items, gmf_user, gmf_item,
                                mlp_user, mlp_item)
    out = _tc_mlp(
        gu, gi, mu, mi,
        W1[:DIM], W1[DIM:], W2, W3, W4,
        Wh[:DIM].reshape(1, DIM), Wh[DIM:],
        b1.reshape(1, -1), b2.reshape(1, -1), b3.reshape(1, -1),
        b4.reshape(1, -1), bh.reshape(1, 1),
    )
    return out.reshape(-1)
